# Initial kernel scaffold; baseline (speedup 1.0000x reference)
import jax
import jax.numpy as jnp
from jax.experimental import pallas as pl


def kernel(x, edge_index, W, b):
    raise NotImplementedError("write your pallas kernel here")


# speedup vs baseline: 9.1823x; 9.1823x over previous
import functools

import jax
import jax.numpy as jnp
from jax import lax
from jax.experimental import pallas as pl
from jax.experimental.pallas import tpu as pltpu
from jax.experimental.pallas import tpu_sc as plsc

_NB = 512

_NC = 2
_NS = 16
_NW = _NC * _NS


def _z_body(x_ref, w2_ref, z_ref):
    xb = x_ref[0]
    z_ref[0] = lax.dot_general(
        xb, w2_ref[...], (((0,), (1,)), ((), ())),
        preferred_element_type=jnp.float32,
        precision=lax.Precision.HIGHEST,
    )


def _z_stage(x3, w2):
    b, c, n = x3.shape
    o = w2.shape[0]
    return pl.pallas_call(
        _z_body,
        grid=(b, n // _NB),
        in_specs=[
            pl.BlockSpec((1, c, _NB), lambda i, j: (i, 0, j)),
            pl.BlockSpec((o, c), lambda i, j: (0, 0)),
        ],
        out_specs=pl.BlockSpec((1, _NB, o), lambda i, j: (i, j, 0)),
        out_shape=jax.ShapeDtypeStruct((b, n, o), jnp.float32),
    )(x3, w2)


def _out_body(x_ref, g_ref, wd_ref, b_ref, o_ref):
    xb = x_ref[0]
    t = lax.dot_general(
        wd_ref[...], xb, (((1,), (0,)), ((), ())),
        preferred_element_type=jnp.float32,
        precision=lax.Precision.HIGHEST,
    )
    gt = g_ref[0].T
    o_ref[0] = jnp.maximum(t + gt + b_ref[...], 0.0)


def _out_stage(x3, g, wd, bias2):
    b, c, n = x3.shape
    o = wd.shape[0]
    return pl.pallas_call(
        _out_body,
        grid=(b, n // _NB),
        in_specs=[
            pl.BlockSpec((1, c, _NB), lambda i, j: (i, 0, j)),
            pl.BlockSpec((1, _NB, o), lambda i, j: (i, j, 0)),
            pl.BlockSpec((o, c), lambda i, j: (0, 0)),
            pl.BlockSpec((o, 1), lambda i, j: (0, 0)),
        ],
        out_specs=pl.BlockSpec((1, o, _NB), lambda i, j: (i, 0, j)),
        out_shape=jax.ShapeDtypeStruct((b, o, n), jnp.float32),
    )(x3, g, wd, bias2)


def _make_gather_max(bn, k, o):
    npw = bn // _NW
    g = 8
    ic = g * k
    nchunk = npw // g

    mesh = plsc.VectorSubcoreMesh(core_axis_name="c", subcore_axis_name="s")

    @functools.partial(
        pl.kernel,
        out_type=jax.ShapeDtypeStruct((bn, o), jnp.float32),
        mesh=mesh,
        compiler_params=pltpu.CompilerParams(use_tc_tiling_on_sc=False),
        scratch_types=[
            pltpu.VMEM((npw * k,), jnp.int32),
            pltpu.VMEM((ic, o), jnp.float32),
            pltpu.VMEM((ic, o), jnp.float32),
            pltpu.VMEM((g, o), jnp.float32),
            pltpu.VMEM((g, o), jnp.float32),
            pltpu.SemaphoreType.DMA,
            pltpu.SemaphoreType.DMA,
            pltpu.SemaphoreType.DMA,
            pltpu.SemaphoreType.DMA,
        ],
    )
    def gather_max(z_hbm, idx_hbm, out_hbm,
                   idx_v, rows0, rows1, out0, out1, g0, g1, s0, s1):
        wid = lax.axis_index("s") * _NC + lax.axis_index("c")
        node_base = wid * npw
        pltpu.sync_copy(idx_hbm.at[pl.ds(node_base * k, npw * k)], idx_v)

        rows = (rows0, rows1)
        outs = (out0, out1)
        gsems = (g0, g1)
        ssems = (s0, s1)

        def fire(c, s):
            pltpu.make_async_copy(
                z_hbm.at[idx_v.at[pl.ds(c * ic, ic)]], rows[s], gsems[s]
            ).start()

        fire(0, 0)
        fire(1, 1)

        def body(i, carry):
            for s in range(2):
                c = 2 * i + s
                pltpu.make_async_copy(
                    z_hbm.at[idx_v.at[pl.ds(0, ic)]], rows[s], gsems[s]
                ).wait()

                @pl.when(i > 0)
                def _():
                    pltpu.make_async_copy(
                        outs[s], out_hbm.at[pl.ds(node_base, g)], ssems[s]
                    ).wait()

                r = rows[s]
                ov = outs[s]
                for n in range(g):
                    row0 = n * k
                    for j in range(o // 16):
                        sl = pl.ds(j * 16, 16)
                        acc = r[row0, sl]
                        for kk in range(1, k):
                            acc = jnp.maximum(acc, r[row0 + kk, sl])
                        ov[n, sl] = acc

                pltpu.make_async_copy(
                    ov, out_hbm.at[pl.ds(node_base + c * g, g)], ssems[s]
                ).start()

                @pl.when(c + 2 < nchunk)
                def _():
                    fire(c + 2, s)
            return carry

        lax.fori_loop(0, nchunk // 2, body, 0)

        pltpu.make_async_copy(
            outs[0], out_hbm.at[pl.ds(node_base, g)], ssems[0]).wait()
        pltpu.make_async_copy(
            outs[1], out_hbm.at[pl.ds(node_base, g)], ssems[1]).wait()

    return gather_max


def kernel(x, edge_index, W, b):
    bsz, c, n, _ = x.shape
    o = W.shape[0]
    k = edge_index.shape[-1]

    x3 = x[..., 0]
    w1 = W[:, :c]
    w2 = W[:, c:]
    wd = w1 - w2
    bias2 = b.reshape(o, 1)

    offs = (jnp.arange(bsz, dtype=jnp.int32) * n)[:, None, None]
    idx_flat = (edge_index[0] + offs).reshape(-1)

    z = _z_stage(x3, w2)
    gmax = _make_gather_max(bsz * n, k, o)
    gathered = gmax(z.reshape(bsz * n, o), idx_flat)
    out = _out_stage(x3, gathered.reshape(bsz, n, o), wd, bias2)
    return out[..., None]


# speedup vs baseline: 14.3555x; 1.5634x over previous
import functools

import jax
import jax.numpy as jnp
from jax import lax
from jax.experimental import pallas as pl
from jax.experimental.pallas import tpu as pltpu
from jax.experimental.pallas import tpu_sc as plsc

_NB = 512

_NC = 2
_NS = 16
_NW = _NC * _NS


def _z_body(x_ref, w_ref, z_ref):
    xb = x_ref[0]
    c = xb.shape[0]
    w2 = w_ref[:, c:]
    z_ref[0] = lax.dot_general(
        xb, w2, (((0,), (1,)), ((), ())),
        preferred_element_type=jnp.float32,
        precision=lax.Precision.HIGHEST,
    )


def _z_stage(x3, w):
    b, c, n = x3.shape
    o = w.shape[0]
    return pl.pallas_call(
        _z_body,
        grid=(b, n // _NB),
        in_specs=[
            pl.BlockSpec((1, c, _NB), lambda i, j: (i, 0, j)),
            pl.BlockSpec((o, 2 * c), lambda i, j: (0, 0)),
        ],
        out_specs=pl.BlockSpec((1, _NB, o), lambda i, j: (i, j, 0)),
        out_shape=jax.ShapeDtypeStruct((b, n, o), jnp.float32),
    )(x3, w)


def _out_body(x_ref, g_ref, w_ref, b_ref, o_ref):
    xb = x_ref[0]
    c = xb.shape[0]
    wd = w_ref[:, :c] - w_ref[:, c:]
    t = lax.dot_general(
        wd, xb, (((1,), (0,)), ((), ())),
        preferred_element_type=jnp.float32,
        precision=lax.Precision.HIGHEST,
    )
    gt = g_ref[0].T
    o_ref[0] = jnp.maximum(t + gt + b_ref[...], 0.0)


def _out_stage(x3, g, w, bias2):
    b, c, n = x3.shape
    o = w.shape[0]
    return pl.pallas_call(
        _out_body,
        grid=(b, n // _NB),
        in_specs=[
            pl.BlockSpec((1, c, _NB), lambda i, j: (i, 0, j)),
            pl.BlockSpec((1, _NB, o), lambda i, j: (i, j, 0)),
            pl.BlockSpec((o, 2 * c), lambda i, j: (0, 0)),
            pl.BlockSpec((o, 1), lambda i, j: (0, 0)),
        ],
        out_specs=pl.BlockSpec((1, o, _NB), lambda i, j: (i, 0, j)),
        out_shape=jax.ShapeDtypeStruct((b, o, n), jnp.float32),
    )(x3, g, w, bias2)


def _make_gather_max(bn, k, o, n_per_batch):
    npw = bn // _NW
    g = 8
    ic = g * k
    nchunk = npw // g

    mesh = plsc.VectorSubcoreMesh(core_axis_name="c", subcore_axis_name="s")

    @functools.partial(
        pl.kernel,
        out_type=jax.ShapeDtypeStruct((bn, o), jnp.float32),
        mesh=mesh,
        compiler_params=pltpu.CompilerParams(use_tc_tiling_on_sc=False),
        scratch_types=[
            pltpu.VMEM((npw * k,), jnp.int32),
            pltpu.VMEM((ic, o), jnp.float32),
            pltpu.VMEM((ic, o), jnp.float32),
            pltpu.VMEM((g, o), jnp.float32),
            pltpu.VMEM((g, o), jnp.float32),
            pltpu.SemaphoreType.DMA,
            pltpu.SemaphoreType.DMA,
            pltpu.SemaphoreType.DMA,
            pltpu.SemaphoreType.DMA,
        ],
    )
    def gather_max(z_hbm, idx_hbm, out_hbm,
                   idx_v, rows0, rows1, out0, out1, g0, g1, s0, s1):
        wid = lax.axis_index("s") * _NC + lax.axis_index("c")
        node_base = wid * npw
        pltpu.sync_copy(idx_hbm.at[0, pl.ds(node_base * k, npw * k)], idx_v)

        off = (node_base // n_per_batch) * n_per_batch
        off_v = jnp.full((16,), off, dtype=jnp.int32)

        def add_off(i, carry):
            sl = pl.ds(i * 16, 16)
            idx_v[sl] = idx_v[sl] + off_v
            return carry

        lax.fori_loop(0, (npw * k) // 16, add_off, 0)

        rows = (rows0, rows1)
        outs = (out0, out1)
        gsems = (g0, g1)
        ssems = (s0, s1)

        def fire(c, s):
            pltpu.make_async_copy(
                z_hbm.at[idx_v.at[pl.ds(c * ic, ic)]], rows[s], gsems[s]
            ).start()

        fire(0, 0)
        fire(1, 1)

        def body(i, carry):
            for s in range(2):
                c = 2 * i + s
                pltpu.make_async_copy(
                    z_hbm.at[idx_v.at[pl.ds(0, ic)]], rows[s], gsems[s]
                ).wait()

                @pl.when(i > 0)
                def _():
                    pltpu.make_async_copy(
                        outs[s], out_hbm.at[pl.ds(node_base, g)], ssems[s]
                    ).wait()

                r = rows[s]
                ov = outs[s]

                def col_body(j, carry):
                    sl = pl.ds(j * 16, 16)
                    for n in range(g):
                        row0 = n * k
                        m0 = jnp.maximum(r[row0 + 0, sl], r[row0 + 1, sl])
                        m1 = jnp.maximum(r[row0 + 2, sl], r[row0 + 3, sl])
                        m2 = jnp.maximum(r[row0 + 4, sl], r[row0 + 5, sl])
                        m3 = jnp.maximum(r[row0 + 6, sl], r[row0 + 7, sl])
                        m0 = jnp.maximum(m0, m1)
                        m2 = jnp.maximum(m2, m3)
                        m0 = jnp.maximum(m0, m2)
                        ov[n, sl] = jnp.maximum(m0, r[row0 + 8, sl])
                    return carry

                lax.fori_loop(0, o // 16, col_body, 0)

                pltpu.make_async_copy(
                    ov, out_hbm.at[pl.ds(node_base + c * g, g)], ssems[s]
                ).start()

                @pl.when(c + 2 < nchunk)
                def _():
                    fire(c + 2, s)
            return carry

        lax.fori_loop(0, nchunk // 2, body, 0)

        pltpu.make_async_copy(
            outs[0], out_hbm.at[pl.ds(node_base, g)], ssems[0]).wait()
        pltpu.make_async_copy(
            outs[1], out_hbm.at[pl.ds(node_base, g)], ssems[1]).wait()

    return gather_max


def kernel(x, edge_index, W, b):
    bsz, c, n, _ = x.shape
    o = W.shape[0]
    k = edge_index.shape[-1]

    x3 = x[..., 0]
    bias2 = b.reshape(o, 1)
    ei_flat = edge_index.reshape(2, bsz * n * k)

    z = _z_stage(x3, W)
    gmax = _make_gather_max(bsz * n, k, o, n)
    gathered = gmax(z.reshape(bsz * n, o), ei_flat)
    out = _out_stage(x3, gathered.reshape(bsz, n, o), W, bias2)
    return out[..., None]


# speedup vs baseline: 15.7094x; 1.0943x over previous
import functools

import jax
import jax.numpy as jnp
from jax import lax
from jax.experimental import pallas as pl
from jax.experimental.pallas import tpu as pltpu
from jax.experimental.pallas import tpu_sc as plsc

_NB = 1024

_NC = 2
_NS = 16
_NW = _NC * _NS


def _z_body(x_ref, w_ref, z_ref):
    xb = x_ref[0]
    c = xb.shape[0]
    w2 = w_ref[:, c:]
    z_ref[...] = lax.dot_general(
        xb, w2, (((0,), (1,)), ((), ())),
        preferred_element_type=jnp.float32,
        precision=lax.Precision.HIGHEST,
    )


def _z_stage(x, w):
    b, c, n = x.shape
    o = w.shape[0]
    nblk = n // _NB
    return pl.pallas_call(
        _z_body,
        grid=(b, nblk),
        in_specs=[
            pl.BlockSpec((1, c, _NB), lambda i, j: (i, 0, j)),
            pl.BlockSpec((o, 2 * c), lambda i, j: (0, 0)),
        ],
        out_specs=pl.BlockSpec((_NB, o), lambda i, j: (i * nblk + j, 0)),
        out_shape=jax.ShapeDtypeStruct((b * n, o), jnp.float32),
    )(x, w)


def _y_body(x_ref, w_ref, b_ref, y_ref):
    xb = x_ref[0]
    c = xb.shape[0]
    wd = w_ref[:, :c] - w_ref[:, c:]
    t = lax.dot_general(
        wd, xb, (((1,), (0,)), ((), ())),
        preferred_element_type=jnp.float32,
        precision=lax.Precision.HIGHEST,
    )
    y_ref[0] = t + b_ref[...]


def _y_stage(x, w, bias2):
    b, c, n = x.shape
    o = w.shape[0]
    return pl.pallas_call(
        _y_body,
        grid=(b, n // _NB),
        in_specs=[
            pl.BlockSpec((1, c, _NB), lambda i, j: (i, 0, j)),
            pl.BlockSpec((o, 2 * c), lambda i, j: (0, 0)),
            pl.BlockSpec((o, 1), lambda i, j: (0, 0)),
        ],
        out_specs=pl.BlockSpec((1, o, _NB), lambda i, j: (i, 0, j)),
        out_shape=jax.ShapeDtypeStruct((b, o, n), jnp.float32),
    )(x, w, bias2)


def _relu_body(y_ref, g_ref, o_ref):
    gt = g_ref[...].T
    o_ref[0] = jnp.maximum(y_ref[0] + gt, 0.0)


def _relu_stage(y, g):
    b, o, n = y.shape
    nblk = n // _NB
    return pl.pallas_call(
        _relu_body,
        grid=(b, nblk),
        in_specs=[
            pl.BlockSpec((1, o, _NB), lambda i, j: (i, 0, j)),
            pl.BlockSpec((_NB, o), lambda i, j: (i * nblk + j, 0)),
        ],
        out_specs=pl.BlockSpec((1, o, _NB), lambda i, j: (i, 0, j)),
        out_shape=jax.ShapeDtypeStruct((b, o, n), jnp.float32),
    )(y, g)


def _make_gather_max(bn, k, o):
    npw = bn // _NW
    g = 8
    ic = g * k
    nchunk = npw // g

    mesh = plsc.VectorSubcoreMesh(core_axis_name="c", subcore_axis_name="s")

    @functools.partial(
        pl.kernel,
        out_type=jax.ShapeDtypeStruct((bn, o), jnp.float32),
        mesh=mesh,
        compiler_params=pltpu.CompilerParams(use_tc_tiling_on_sc=False),
        scratch_types=[
            pltpu.VMEM((npw * k,), jnp.int32),
            pltpu.VMEM((ic, o), jnp.float32),
            pltpu.VMEM((ic, o), jnp.float32),
            pltpu.VMEM((g, o), jnp.float32),
            pltpu.VMEM((g, o), jnp.float32),
            pltpu.SemaphoreType.DMA,
            pltpu.SemaphoreType.DMA,
            pltpu.SemaphoreType.DMA,
            pltpu.SemaphoreType.DMA,
        ],
    )
    def gather_max(z_hbm, idx_hbm, out_hbm,
                   idx_v, rows0, rows1, out0, out1, g0, g1, s0, s1):
        wid = lax.axis_index("s") * _NC + lax.axis_index("c")
        node_base = wid * npw
        pltpu.sync_copy(idx_hbm.at[pl.ds(node_base * k, npw * k)], idx_v)

        rows = (rows0, rows1)
        outs = (out0, out1)
        gsems = (g0, g1)
        ssems = (s0, s1)

        def fire(c, s):
            pltpu.make_async_copy(
                z_hbm.at[idx_v.at[pl.ds(c * ic, ic)]], rows[s], gsems[s]
            ).start()

        fire(0, 0)
        fire(1, 1)

        def body(i, carry):
            for s in range(2):
                c = 2 * i + s
                pltpu.make_async_copy(
                    z_hbm.at[idx_v.at[pl.ds(0, ic)]], rows[s], gsems[s]
                ).wait()

                @pl.when(i > 0)
                def _():
                    pltpu.make_async_copy(
                        outs[s], out_hbm.at[pl.ds(node_base, g)], ssems[s]
                    ).wait()

                r = rows[s]
                ov = outs[s]

                def col_body(j, carry2):
                    sl = pl.ds(j * 16, 16)
                    for n in range(g):
                        row0 = n * k
                        m0 = jnp.maximum(r[row0 + 0, sl], r[row0 + 1, sl])
                        m1 = jnp.maximum(r[row0 + 2, sl], r[row0 + 3, sl])
                        m2 = jnp.maximum(r[row0 + 4, sl], r[row0 + 5, sl])
                        m3 = jnp.maximum(r[row0 + 6, sl], r[row0 + 7, sl])
                        m0 = jnp.maximum(m0, m1)
                        m2 = jnp.maximum(m2, m3)
                        m0 = jnp.maximum(m0, m2)
                        ov[n, sl] = jnp.maximum(m0, r[row0 + 8, sl])
                    return carry2

                lax.fori_loop(0, o // 16, col_body, 0)

                pltpu.make_async_copy(
                    ov, out_hbm.at[pl.ds(node_base + c * g, g)], ssems[s]
                ).start()

                @pl.when(c + 2 < nchunk)
                def _():
                    fire(c + 2, s)
            return carry

        lax.fori_loop(0, nchunk // 2, body, 0)

        pltpu.make_async_copy(
            outs[0], out_hbm.at[pl.ds(node_base, g)], ssems[0]).wait()
        pltpu.make_async_copy(
            outs[1], out_hbm.at[pl.ds(node_base, g)], ssems[1]).wait()

    return gather_max


def kernel(x, edge_index, W, b):
    bsz, c, n, _ = x.shape
    o = W.shape[0]
    k = edge_index.shape[-1]

    bias2 = b.reshape(o, 1)

    offs = (jnp.arange(bsz, dtype=jnp.int32) * n)[:, None, None]
    idx_flat = (edge_index[0] + offs).reshape(-1)

    x3 = x[..., 0]
    z = _z_stage(x3, W)
    gmax = _make_gather_max(bsz * n, k, o)
    gathered = gmax(z, idx_flat)
    y = _y_stage(x3, W, bias2)
    return _relu_stage(y, gathered)[..., None]


# speedup vs baseline: 17.3652x; 1.1054x over previous
import functools

import numpy as np
import jax
import jax.numpy as jnp
from jax import lax
from jax.experimental import pallas as pl
from jax.experimental.pallas import tpu as pltpu
from jax.experimental.pallas import tpu_sc as plsc

_NB = 1024

_NC = 2
_NS = 16
_NW = _NC * _NS


def _z_body(x_ref, w_ref, z_ref):
    xb = x_ref[0]
    c = xb.shape[0]
    w2 = w_ref[:, c:]
    z_ref[...] = lax.dot_general(
        xb, w2, (((0,), (1,)), ((), ())),
        preferred_element_type=jnp.float32,
        precision=lax.Precision.HIGHEST,
    )


def _z_stage(x, w):
    b, c, n = x.shape
    opad = w.shape[0]
    nblk = n // _NB
    return pl.pallas_call(
        _z_body,
        grid=(b, nblk),
        in_specs=[
            pl.BlockSpec((1, c, _NB), lambda i, j: (i, 0, j)),
            pl.BlockSpec((opad, 2 * c), lambda i, j: (0, 0)),
        ],
        out_specs=pl.BlockSpec((_NB, opad), lambda i, j: (i * nblk + j, 0)),
        out_shape=jax.ShapeDtypeStruct((b * n, opad), jnp.float32),
    )(x, w)


def _y_body(x_ref, w_ref, b_ref, y_ref):
    xb = x_ref[0]
    c = xb.shape[0]
    wd = w_ref[:, :c] - w_ref[:, c:]
    t = lax.dot_general(
        wd, xb, (((1,), (0,)), ((), ())),
        preferred_element_type=jnp.float32,
        precision=lax.Precision.HIGHEST,
    )
    y_ref[0] = t + b_ref[...]


def _y_stage(x, w, bias2):
    b, c, n = x.shape
    o = w.shape[0]
    return pl.pallas_call(
        _y_body,
        grid=(b, n // _NB),
        in_specs=[
            pl.BlockSpec((1, c, _NB), lambda i, j: (i, 0, j)),
            pl.BlockSpec((o, 2 * c), lambda i, j: (0, 0)),
            pl.BlockSpec((o, 1), lambda i, j: (0, 0)),
        ],
        out_specs=pl.BlockSpec((1, o, _NB), lambda i, j: (i, 0, j)),
        out_shape=jax.ShapeDtypeStruct((b, o, n), jnp.float32),
    )(x, w, bias2)


def _relu_body(y_ref, g_ref, o_ref):
    o = y_ref.shape[1]
    gt = g_ref[:, :o].T
    o_ref[0] = jnp.maximum(y_ref[0] + gt, 0.0)


def _relu_stage(y, g):
    b, o, n = y.shape
    opad = g.shape[1]
    nblk = n // _NB
    return pl.pallas_call(
        _relu_body,
        grid=(b, nblk),
        in_specs=[
            pl.BlockSpec((1, o, _NB), lambda i, j: (i, 0, j)),
            pl.BlockSpec((_NB, opad), lambda i, j: (i * nblk + j, 0)),
        ],
        out_specs=pl.BlockSpec((1, o, _NB), lambda i, j: (i, 0, j)),
        out_shape=jax.ShapeDtypeStruct((b, o, n), jnp.float32),
    )(y, g)


def _make_gather_max(bn, k, o, opad):
    npw = bn // _NW
    g = 8
    ic = g * k
    nchunk = npw // g

    mesh = plsc.VectorSubcoreMesh(core_axis_name="c", subcore_axis_name="s")

    @functools.partial(
        pl.kernel,
        out_type=jax.ShapeDtypeStruct((bn, opad), jnp.float32),
        mesh=mesh,
        compiler_params=pltpu.CompilerParams(use_tc_tiling_on_sc=True),
        scratch_types=[
            pltpu.VMEM((npw * k,), jnp.int32),
            pltpu.VMEM((ic, opad), jnp.float32),
            pltpu.VMEM((ic, opad), jnp.float32),
            pltpu.VMEM((g, opad), jnp.float32),
            pltpu.VMEM((g, opad), jnp.float32),
            pltpu.SemaphoreType.DMA,
            pltpu.SemaphoreType.DMA,
            pltpu.SemaphoreType.DMA,
            pltpu.SemaphoreType.DMA,
        ],
    )
    def gather_max(z_hbm, idx_hbm, out_hbm,
                   idx_v, rows0, rows1, out0, out1, g0, g1, s0, s1):
        wid = lax.axis_index("s") * _NC + lax.axis_index("c")
        node_base = wid * npw
        pltpu.sync_copy(idx_hbm.at[pl.ds(node_base * k, npw * k)], idx_v)

        rows = (rows0, rows1)
        outs = (out0, out1)
        gsems = (g0, g1)
        ssems = (s0, s1)

        def fire(c, s):
            pltpu.make_async_copy(
                z_hbm.at[idx_v.at[pl.ds(c * ic, ic)]], rows[s], gsems[s]
            ).start()

        fire(0, 0)
        fire(1, 1)

        def body(i, carry):
            for s in range(2):
                c = 2 * i + s
                pltpu.make_async_copy(
                    z_hbm.at[idx_v.at[pl.ds(0, ic)]], rows[s], gsems[s]
                ).wait()

                @pl.when(i > 0)
                def _():
                    pltpu.make_async_copy(
                        outs[s], out_hbm.at[pl.ds(node_base, g)], ssems[s]
                    ).wait()

                r = rows[s]
                ov = outs[s]

                def col_body(j, carry2):
                    sl = pl.ds(j * 16, 16)
                    for n in range(g):
                        row0 = n * k
                        m0 = jnp.maximum(r[row0 + 0, sl], r[row0 + 1, sl])
                        m1 = jnp.maximum(r[row0 + 2, sl], r[row0 + 3, sl])
                        m2 = jnp.maximum(r[row0 + 4, sl], r[row0 + 5, sl])
                        m3 = jnp.maximum(r[row0 + 6, sl], r[row0 + 7, sl])
                        m0 = jnp.maximum(m0, m1)
                        m2 = jnp.maximum(m2, m3)
                        m0 = jnp.maximum(m0, m2)
                        ov[n, sl] = jnp.maximum(m0, r[row0 + 8, sl])
                    return carry2

                lax.fori_loop(0, o // 16, col_body, 0)

                pltpu.make_async_copy(
                    ov, out_hbm.at[pl.ds(node_base + c * g, g)], ssems[s]
                ).start()

                @pl.when(c + 2 < nchunk)
                def _():
                    fire(c + 2, s)
            return carry

        lax.fori_loop(0, nchunk // 2, body, 0)

        pltpu.make_async_copy(
            outs[0], out_hbm.at[pl.ds(node_base, g)], ssems[0]).wait()
        pltpu.make_async_copy(
            outs[1], out_hbm.at[pl.ds(node_base, g)], ssems[1]).wait()

    return gather_max


def kernel(x, edge_index, W, b):
    bsz, c, n, _ = x.shape
    o = W.shape[0]
    k = edge_index.shape[-1]

    bias2 = b.reshape(o, 1)
    opad = 256

    offs = np.arange(bsz, dtype=np.int32).reshape(bsz, 1, 1) * n
    idx_flat = (edge_index[0] + jnp.asarray(offs)).reshape(-1)

    w_pad = jnp.pad(W, ((0, opad - o), (0, 0)))

    x3 = x.reshape(bsz, c, n)
    z = _z_stage(x3, w_pad)
    gmax = _make_gather_max(bsz * n, k, o, opad)
    gathered = gmax(z, idx_flat)
    y = _y_stage(x3, W, bias2)
    return _relu_stage(y, gathered)[..., None]
